# baseline (speedup 1.0000x reference)
import jax
import jax.numpy as jnp
from jax.experimental import pallas as pl

H = 64


def _ln(t, g, b):
    m = t.mean(-1, keepdims=True)
    v = ((t - m) ** 2).mean(-1, keepdims=True)
    return (t - m) / jnp.sqrt(v + 1e-5) * g + b


def _bn(x, g, b):
    m = x.mean(0)
    v = ((x - m) ** 2).mean(0)
    return (x - m) / jnp.sqrt(v + 1e-5) * g + b


def _pad_rows(x, blk):
    n = x.shape[0]
    npad = ((n + blk - 1) // blk) * blk
    if npad != n:
        x = jnp.pad(x, ((0, npad - n), (0, 0)))
    return x, npad


def _row_spec(blk, m):
    return pl.BlockSpec((blk, m), lambda i: (i, 0))


def _full_spec(shape):
    nd = len(shape)
    return pl.BlockSpec(shape, lambda i: (0,) * nd)


def _mm_body(x_ref, w_ref, b_ref, o_ref):
    o_ref[...] = jnp.dot(x_ref[...], w_ref[...],
                         preferred_element_type=jnp.float32) + b_ref[...]


def _matmul(x, W, b, blk=2048):
    n, k = x.shape
    x, npad = _pad_rows(x, blk)
    m = W.shape[1]
    out = pl.pallas_call(
        _mm_body,
        grid=(npad // blk,),
        in_specs=[_row_spec(blk, k), _full_spec(W.shape), _full_spec(b.shape)],
        out_specs=_row_spec(blk, m),
        out_shape=jax.ShapeDtypeStruct((npad, m), jnp.float32),
    )(x, W, b)
    return out[:n]


def _mm4_body(h_ref, wq_ref, bq_ref, wk_ref, bk_ref, wv_ref, bv_ref,
              ws_ref, bs_ref, q_ref, k_ref, v_ref, s_ref):
    hh = h_ref[...]
    q_ref[...] = jnp.dot(hh, wq_ref[...], preferred_element_type=jnp.float32) + bq_ref[...]
    k_ref[...] = jnp.dot(hh, wk_ref[...], preferred_element_type=jnp.float32) + bk_ref[...]
    v_ref[...] = jnp.dot(hh, wv_ref[...], preferred_element_type=jnp.float32) + bv_ref[...]
    s_ref[...] = jnp.dot(hh, ws_ref[...], preferred_element_type=jnp.float32) + bs_ref[...]


def _qkvs(h, tp, blk=2048):
    n = h.shape[0]
    h, npad = _pad_rows(h, blk)
    outs = pl.pallas_call(
        _mm4_body,
        grid=(npad // blk,),
        in_specs=[_row_spec(blk, H)] + [
            _full_spec(s) for s in [tp['Wq'].shape, tp['bq'].shape,
                                    tp['Wk'].shape, tp['bk'].shape,
                                    tp['Wv'].shape, tp['bv'].shape,
                                    tp['Ws'].shape, tp['bs'].shape]],
        out_specs=[_row_spec(blk, H)] * 4,
        out_shape=[jax.ShapeDtypeStruct((npad, H), jnp.float32)] * 4,
    )(h, tp['Wq'], tp['bq'], tp['Wk'], tp['bk'], tp['Wv'], tp['bv'],
      tp['Ws'], tp['bs'])
    return [o[:n] for o in outs]


def kernel(x, edge_index, edge_attr, batch, nA, nB, system_size, params):
    N = x.shape[0]
    E = edge_index.shape[1]
    B = nA.shape[0]
    src = edge_index[0]
    dst = edge_index[1]

    pe = params['node_enc']
    h = jax.nn.silu(_ln(_matmul(x, pe['W'], pe['b']), pe['g'], pe['be']))
    qe = params['edge_enc']
    e = jax.nn.silu(_ln(_matmul(edge_attr, qe['W'], qe['b']), qe['g'], qe['be']))

    for i in range(8):
        if i % 2 == 0:
            gp = params['gine'][i // 2]
            ee = _matmul(e, gp['We'], gp['bee'])
            msg = jax.nn.relu(h[src] + ee)
            aggr = jax.ops.segment_sum(msg, dst, num_segments=N)
            t = jax.nn.silu(_ln(_matmul(h + aggr, gp['W1'], gp['b1']),
                                gp['g1'], gp['bn1']))
            hn = _matmul(t, gp['W2'], gp['b2'])
        else:
            tp = params['tr'][i // 2]
            q, k, v, x_r = _qkvs(h, tp)
            q = q.reshape(N, 4, H // 4)
            k = k.reshape(N, 4, H // 4)
            v = v.reshape(N, 4, H // 4)
            ee = _matmul(e, tp['We'], tp['bee']).reshape(E, 4, H // 4)
            alpha = (q[dst] * (k[src] + ee)).sum(-1) / jnp.sqrt(float(H // 4))
            m = jax.ops.segment_max(alpha, dst, num_segments=N)
            m = jnp.where(jnp.isfinite(m), m, 0.0)
            al = jnp.exp(alpha - m[dst])
            den = jax.ops.segment_sum(al, dst, num_segments=N)
            al = al / (den[dst] + 1e-16)
            msg = al[:, :, None] * (v[src] + ee)
            out = jax.ops.segment_sum(msg.reshape(E, H), dst, num_segments=N)
            beta = jax.nn.sigmoid(jnp.concatenate([out, x_r, out - x_r], axis=1) @ tp['Wb'])
            hn = beta * x_r + (1.0 - beta) * out
        bnp_ = params['bn'][i]
        hn = _bn(hn, bnp_['g'], bnp_['b'])
        h = h + hn

    lp = params['lstm']
    q_star = jnp.zeros((B, 2 * H), jnp.float32)
    hs = jnp.zeros((B, H), jnp.float32)
    cs = jnp.zeros((B, H), jnp.float32)
    for _ in range(4):
        gates = q_star @ lp['Wih'] + lp['bih'] + hs @ lp['Whh'] + lp['bhh']
        ii, ff, gg, oo = jnp.split(gates, 4, axis=1)
        cs = jax.nn.sigmoid(ff) * cs + jax.nn.sigmoid(ii) * jnp.tanh(gg)
        hs = jax.nn.sigmoid(oo) * jnp.tanh(cs)
        en = (h * hs[batch]).sum(-1)
        mm = jax.ops.segment_max(en, batch, num_segments=B)
        mm = jnp.where(jnp.isfinite(mm), mm, 0.0)
        ex = jnp.exp(en - mm[batch])
        dn = jax.ops.segment_sum(ex, batch, num_segments=B)
        a = ex / (dn[batch] + 1e-16)
        r = jax.ops.segment_sum(a[:, None] * h, batch, num_segments=B)
        q_star = jnp.concatenate([hs, r], axis=1)

    gp2 = params['glob']
    gfeat = jnp.stack([nA[:, 0] / (system_size[:, 0] + 1e-10),
                       nB[:, 0] / (system_size[:, 0] + 1e-10)], axis=1)
    gf = jax.nn.silu(_ln(gfeat @ gp2['W1'] + gp2['b1'], gp2['g'], gp2['be'])) @ gp2['W2'] + gp2['b2']
    fp = params['fin']
    c = jnp.concatenate([q_star, gf], axis=1)
    t = jax.nn.silu(_ln(c @ fp['W1'] + fp['b1'], fp['g1'], fp['be1']))
    t = jax.nn.silu(_ln(t @ fp['W2'] + fp['b2'], fp['g2'], fp['be2']))
    t = t @ fp['W3'] + fp['b3']
    return t[:, 0]
